# Initial kernel scaffold; baseline (speedup 1.0000x reference)
import jax
import jax.numpy as jnp
from jax.experimental import pallas as pl


def kernel(predictions, targets):
    raise NotImplementedError("write your pallas kernel here")


# speedup vs baseline: 6.3516x; 6.3516x over previous
import functools
import math

import jax
import jax.numpy as jnp
from jax.experimental import pallas as pl
from jax.experimental.pallas import tpu as pltpu

_N = 1_000_000
_K = 100
_LANES = 128
_ROWS = 7816
_PADDED = _ROWS * _LANES
_NEG_INF = float("-inf")


def _ndcg_kernel(pred_ref, targ_ref, out_ref, pred_scratch, targ_scratch):
    pred_scratch[:, :] = pred_ref[:, :]
    targ_scratch[:, :] = targ_ref[:, :]

    flat_iota = (
        jax.lax.broadcasted_iota(jnp.int32, (_ROWS, _LANES), 0) * _LANES
        + jax.lax.broadcasted_iota(jnp.int32, (_ROWS, _LANES), 1)
    )
    col_iota = jax.lax.broadcasted_iota(jnp.int32, (1, _LANES), 1)
    int_max = jnp.int32(2147483647)


    def pred_body(i, dcg):
        x = pred_scratch[:, :]
        m = jnp.max(x)
        idx = jnp.min(jnp.where(x == m, flat_iota, int_max))
        r = idx // _LANES
        c = idx % _LANES
        lane_mask = col_iota == c
        trow = targ_ref[pl.ds(r, 1), :]
        rel = jnp.sum(jnp.where(lane_mask, trow, 0.0))
        xrow = pred_scratch[pl.ds(r, 1), :]
        pred_scratch[pl.ds(r, 1), :] = jnp.where(lane_mask, _NEG_INF, xrow)
        inv_disc = 1.0 / jnp.log2(i.astype(jnp.float32) + 2.0)
        return dcg + rel * inv_disc

    dcg = jax.lax.fori_loop(0, _K, pred_body, jnp.float32(0.0))

    def targ_body(i, idcg):
        t = targ_scratch[:, :]
        m = jnp.max(t)
        idx = jnp.min(jnp.where(t == m, flat_iota, int_max))
        r = idx // _LANES
        c = idx % _LANES
        lane_mask = col_iota == c
        trow = targ_scratch[pl.ds(r, 1), :]
        targ_scratch[pl.ds(r, 1), :] = jnp.where(lane_mask, _NEG_INF, trow)
        inv_disc = 1.0 / jnp.log2(i.astype(jnp.float32) + 2.0)
        return idcg + m * inv_disc

    idcg = jax.lax.fori_loop(0, _K, targ_body, jnp.float32(0.0))

    safe_idcg = jnp.where(idcg == 0.0, 1.0, idcg)
    ndcg = jnp.where(idcg == 0.0, 0.0, dcg / safe_idcg)
    out_ref[:, :] = jnp.broadcast_to(1.0 - ndcg, (1, 1))


@jax.jit
def kernel(predictions, targets):
    if predictions.ndim == 2:
        predictions = jnp.squeeze(predictions, axis=1)
    if targets.ndim == 2:
        targets = jnp.squeeze(targets, axis=1)
    pad = _PADDED - _N
    pred2 = jnp.pad(predictions, (0, pad), constant_values=_NEG_INF)
    targ2 = jnp.pad(targets, (0, pad), constant_values=_NEG_INF)
    pred2 = pred2.reshape(_ROWS, _LANES)
    targ2 = targ2.reshape(_ROWS, _LANES)

    out = pl.pallas_call(
        _ndcg_kernel,
        out_shape=jax.ShapeDtypeStruct((1, 1), jnp.float32),
        scratch_shapes=[
            pltpu.VMEM((_ROWS, _LANES), jnp.float32),
            pltpu.VMEM((_ROWS, _LANES), jnp.float32),
        ],
    )(pred2, targ2)
    return out[0, 0]


# speedup vs baseline: 20.2827x; 3.1933x over previous
import jax
import jax.numpy as jnp
from jax.experimental import pallas as pl
from jax.experimental.pallas import tpu as pltpu

_N = 1_000_000
_K = 100
_LANES = 128
_GROUP = 128
_NGROUPS = 64
_ROWS = _GROUP * _NGROUPS
_PADDED = _ROWS * _LANES
_NEG_INF = float("-inf")


def _ndcg_kernel(pred_ref, targ_ref, out_ref, pred_scratch, targ_scratch,
                 tp_ref, tt_ref):
    pred_scratch[:, :] = pred_ref[:, :]
    targ_scratch[:, :] = targ_ref[:, :]

    def build(g, _):
        tp_ref[pl.ds(g, 1), :] = jnp.max(
            pred_scratch[pl.ds(g * _GROUP, _GROUP), :], axis=0, keepdims=True)
        tt_ref[pl.ds(g, 1), :] = jnp.max(
            targ_scratch[pl.ds(g * _GROUP, _GROUP), :], axis=0, keepdims=True)
        return 0
    jax.lax.fori_loop(0, _NGROUPS, build, 0)

    tile_iota = (
        jax.lax.broadcasted_iota(jnp.int32, (_NGROUPS, _LANES), 0) * _LANES
        + jax.lax.broadcasted_iota(jnp.int32, (_NGROUPS, _LANES), 1)
    )
    local_iota = (
        jax.lax.broadcasted_iota(jnp.int32, (_GROUP, _LANES), 0) * _LANES
        + jax.lax.broadcasted_iota(jnp.int32, (_GROUP, _LANES), 1)
    )
    col_iota = jax.lax.broadcasted_iota(jnp.int32, (1, _LANES), 1)
    int_max = jnp.int32(2147483647)

    def extract(data_scratch, t_ref):
        tbl = t_ref[:, :]
        m = jnp.max(tbl)
        tidx = jnp.min(jnp.where(tbl == m, tile_iota, int_max))
        g = tidx // _LANES
        block = data_scratch[pl.ds(g * _GROUP, _GROUP), :]
        lf = jnp.min(jnp.where(block == m, local_iota, int_max))
        r = g * _GROUP + lf // _LANES
        c = lf % _LANES
        lane_mask = col_iota == c
        row = data_scratch[pl.ds(r, 1), :]
        data_scratch[pl.ds(r, 1), :] = jnp.where(lane_mask, _NEG_INF, row)
        t_ref[pl.ds(g, 1), :] = jnp.max(
            data_scratch[pl.ds(g * _GROUP, _GROUP), :], axis=0, keepdims=True)
        return m, r, c

    def pred_body(i, dcg):
        _, r, c = extract(pred_scratch, tp_ref)
        trow = targ_ref[pl.ds(r, 1), :]
        rel = jnp.sum(jnp.where(col_iota == c, trow, 0.0))
        inv_disc = 1.0 / jnp.log2(i.astype(jnp.float32) + 2.0)
        return dcg + rel * inv_disc

    dcg = jax.lax.fori_loop(0, _K, pred_body, jnp.float32(0.0))

    def targ_body(i, idcg):
        m, _, _ = extract(targ_scratch, tt_ref)
        inv_disc = 1.0 / jnp.log2(i.astype(jnp.float32) + 2.0)
        return idcg + m * inv_disc

    idcg = jax.lax.fori_loop(0, _K, targ_body, jnp.float32(0.0))

    safe_idcg = jnp.where(idcg == 0.0, 1.0, idcg)
    ndcg = jnp.where(idcg == 0.0, 0.0, dcg / safe_idcg)
    out_ref[:, :] = jnp.broadcast_to(1.0 - ndcg, (1, 1))


@jax.jit
def kernel(predictions, targets):
    if predictions.ndim == 2:
        predictions = jnp.squeeze(predictions, axis=1)
    if targets.ndim == 2:
        targets = jnp.squeeze(targets, axis=1)
    pad = _PADDED - _N
    pred2 = jnp.pad(predictions, (0, pad), constant_values=_NEG_INF)
    targ2 = jnp.pad(targets, (0, pad), constant_values=_NEG_INF)
    pred2 = pred2.reshape(_ROWS, _LANES)
    targ2 = targ2.reshape(_ROWS, _LANES)

    out = pl.pallas_call(
        _ndcg_kernel,
        out_shape=jax.ShapeDtypeStruct((1, 1), jnp.float32),
        scratch_shapes=[
            pltpu.VMEM((_ROWS, _LANES), jnp.float32),
            pltpu.VMEM((_ROWS, _LANES), jnp.float32),
            pltpu.VMEM((_NGROUPS, _LANES), jnp.float32),
            pltpu.VMEM((_NGROUPS, _LANES), jnp.float32),
        ],
    )(pred2, targ2)
    return out[0, 0]
